# baseline (speedup 1.0000x reference)
import jax
import jax.numpy as jnp
from jax.experimental import pallas as pl
from jax.experimental.pallas import tpu as pltpu

_N = 100000
_H = 512
_D = 16
_TN = 2000
_T = _N // _TN
_CT = 20


def _hgnn_body(adj_ref, et_ref, outt_ref, cache, lat, lat2):
    p = pl.program_id(0)
    i = pl.program_id(1)

    @pl.when(p == 0)
    def _phase_a():
        @pl.when(i == 0)
        def _():
            lat[...] = jnp.zeros_like(lat)

        a = adj_ref[...]
        e = et_ref[0]
        lat[...] += jax.lax.dot_general(
            e, a, (((1,), (0,)), ((), ())),
            preferred_element_type=jnp.float32)

        @pl.when(i < _CT)
        def _():
            cache[pl.ds(i * _TN, _TN), :] = a.astype(jnp.bfloat16)

    @pl.when(p == 1)
    def _phase_b():
        @pl.when(i == 0)
        def _():
            lat2[...] = lat[...].T.astype(jnp.bfloat16)

        lb = lat2[...]

        @pl.when(i < _CT)
        def _cached():
            c = cache[pl.ds(i * _TN, _TN), :]
            r = jax.lax.dot_general(
                c, lb, (((1,), (0,)), ((), ())),
                preferred_element_type=jnp.float32)
            outt_ref[0] = r.T

        @pl.when(i >= _CT)
        def _streamed():
            a = adj_ref[...].astype(jnp.bfloat16)
            r = jax.lax.dot_general(
                a, lb, (((1,), (0,)), ((), ())),
                preferred_element_type=jnp.float32)
            outt_ref[0] = r.T


def kernel(adj, embeds):
    e3 = embeds.T.reshape(_D, _T, _TN).swapaxes(0, 1)
    out3 = pl.pallas_call(
        _hgnn_body,
        grid=(2, _T),
        in_specs=[
            pl.BlockSpec(
                (_TN, _H),
                lambda p, i: (jnp.where(p == 0, i, jnp.where(i < _CT, _T - 1, i)), 0)),
            pl.BlockSpec((1, _D, _TN), lambda p, i: (jnp.where(p == 0, i, 0), 0, 0)),
        ],
        out_specs=pl.BlockSpec(
            (1, _D, _TN), lambda p, i: (jnp.where(p == 0, 0, i), 0, 0)),
        out_shape=jax.ShapeDtypeStruct((_T, _D, _TN), jnp.float32),
        scratch_shapes=[
            pltpu.VMEM((_CT * _TN, _H), jnp.bfloat16),
            pltpu.VMEM((_D, _H), jnp.float32),
            pltpu.VMEM((_H, _D), jnp.bfloat16),
        ],
        compiler_params=pltpu.CompilerParams(
            dimension_semantics=("arbitrary", "arbitrary"),
            vmem_limit_bytes=64 * 1024 * 1024,
        ),
    )(adj, e3)
    return out3.swapaxes(0, 1).reshape(_D, _N).T
